# baseline (speedup 1.0000x reference)
import functools

import jax
import jax.numpy as jnp
from jax import lax
from jax.experimental import pallas as pl
from jax.experimental.pallas import tpu as pltpu
from jax.experimental.pallas import tpu_sc as plsc

N = 10000
E = 320000
NC = 2
NS = 16
CHUNK = 80
NCH_ES = 125
NCH_FS = 250
NPAD = 10240
ROWS_PT = NPAD // NS

_mesh = plsc.VectorSubcoreMesh(core_axis_name="c", subcore_axis_name="s")
_sc_params = pltpu.CompilerParams(use_tc_tiling_on_sc=False)

_RING = 8


def _fill_vmem(buf, rows, width, value):
    vec = jnp.full((16,), value, jnp.float32)
    for cb in range(width // 16):
        def wr(i, carry):
            buf[i, pl.ds(cb * 16, 16)] = vec
            return carry
        lax.fori_loop(0, rows, wr, 0)


def _zero_acc_slice(zsrc, acc, s):
    for t in range(ROWS_PT // CHUNK):
        pltpu.sync_copy(zsrc, acc.at[pl.ds(s * ROWS_PT + t * CHUNK, CHUNK)])


@functools.partial(
    pl.kernel, mesh=_mesh, compiler_params=_sc_params,
    out_type=jax.ShapeDtypeStruct((NC, NPAD, 16), jnp.float32),
    scratch_types=[
        pltpu.VMEM((NCH_ES, CHUNK), jnp.int32),
        pltpu.VMEM((CHUNK, 16), jnp.float32),
        pltpu.VMEM_SHARED((NPAD, 16), jnp.float32),
        pltpu.SemaphoreType.DMA,
    ],
)
def _deg_kernel(eidx_hbm, out_hbm, idx_v, ones_v, acc, sem):
    c = lax.axis_index("c")
    s = lax.axis_index("s")
    w = c * NS + s
    _fill_vmem(ones_v, CHUNK, 16, 0.0)
    _zero_acc_slice(ones_v, acc, s)
    _fill_vmem(ones_v, CHUNK, 16, 1.0)
    pltpu.sync_copy(eidx_hbm.at[1, w], idx_v)
    plsc.subcore_barrier()

    def fire(j, carry):
        pltpu.async_copy(ones_v, acc.at[idx_v.at[j]], sem, add=True)
        return carry

    lax.fori_loop(0, NCH_ES, fire, 0)

    def drain(j, carry):
        pltpu.make_async_copy(ones_v, acc.at[idx_v.at[j]], sem).wait()
        return carry

    lax.fori_loop(0, NCH_ES, drain, 0)
    plsc.subcore_barrier()
    pltpu.sync_copy(acc.at[pl.ds(s * ROWS_PT, ROWS_PT)],
                    out_hbm.at[c, pl.ds(s * ROWS_PT, ROWS_PT)])


def _agg_body(h_tbl, src_v, dst_v, rows_v, acc, gsem, ssem, nchunk, ring,
              lag):
    def gather_start(j, b):
        pltpu.async_copy(h_tbl.at[src_v.at[j]], rows_v.at[b], gsem.at[b])

    def gather_wait(j, b):
        pltpu.make_async_copy(h_tbl.at[src_v.at[j]], rows_v.at[b],
                              gsem.at[b]).wait()

    def scat_start(j, b):
        pltpu.async_copy(rows_v.at[b], acc.at[dst_v.at[j]], ssem.at[b],
                         add=True)

    def scat_wait(j, b):
        pltpu.make_async_copy(rows_v.at[b], acc.at[dst_v.at[j]],
                              ssem.at[b]).wait()

    for k in range(ring):
        gather_start(k, k)

    def body(j, carry):
        b = lax.rem(j, ring)
        gather_wait(j, b)

        @pl.when((j >= lag) & (j - lag + ring < nchunk))
        def _():
            jp = j - lag
            bp = lax.rem(jp, ring)
            scat_wait(jp, bp)
            gather_start(jp + ring, bp)

        scat_start(j, b)
        return carry

    lax.fori_loop(0, nchunk, body, 0)
    for k in range(nchunk - ring, nchunk):
        scat_wait(k, k % ring)


@functools.partial(
    pl.kernel, mesh=_mesh, compiler_params=_sc_params,
    out_type=jax.ShapeDtypeStruct((NC, NPAD, 64), jnp.float32),
    scratch_types=[
        pltpu.VMEM((NCH_FS, CHUNK), jnp.int32),
        pltpu.VMEM((NCH_FS, CHUNK), jnp.int32),
        pltpu.VMEM((_RING, CHUNK, 64), jnp.float32),
        pltpu.VMEM_SHARED((NPAD, 64), jnp.float32),
        pltpu.SemaphoreType.DMA((_RING,)),
        pltpu.SemaphoreType.DMA((_RING,)),
    ],
)
def _agg128(h_hbm, eidx_hbm, out_hbm,
            src_v, dst_v, rows_v, acc, gsem, ssem):
    c = lax.axis_index("c")
    s = lax.axis_index("s")
    _fill_vmem(rows_v.at[0], CHUNK, 64, 0.0)
    _zero_acc_slice(rows_v.at[0], acc, s)
    pltpu.sync_copy(eidx_hbm.at[0, 2 * s], src_v.at[pl.ds(0, NCH_ES)])
    pltpu.sync_copy(eidx_hbm.at[0, 2 * s + 1], src_v.at[pl.ds(NCH_ES, NCH_ES)])
    pltpu.sync_copy(eidx_hbm.at[1, 2 * s], dst_v.at[pl.ds(0, NCH_ES)])
    pltpu.sync_copy(eidx_hbm.at[1, 2 * s + 1], dst_v.at[pl.ds(NCH_ES, NCH_ES)])
    plsc.subcore_barrier()
    _agg_body(h_hbm.at[c], src_v, dst_v, rows_v, acc, gsem, ssem, NCH_FS,
              _RING, 6)
    plsc.subcore_barrier()
    pltpu.sync_copy(acc.at[pl.ds(s * ROWS_PT, ROWS_PT)],
                    out_hbm.at[c, pl.ds(s * ROWS_PT, ROWS_PT)])


@functools.partial(
    pl.kernel, mesh=_mesh, compiler_params=_sc_params,
    out_type=jax.ShapeDtypeStruct((NC, NPAD, 16), jnp.float32),
    scratch_types=[
        pltpu.VMEM((NCH_ES, CHUNK), jnp.int32),
        pltpu.VMEM((NCH_ES, CHUNK), jnp.int32),
        pltpu.VMEM((_RING, CHUNK, 16), jnp.float32),
        pltpu.VMEM_SHARED((NPAD, 16), jnp.float32),
        pltpu.SemaphoreType.DMA((_RING,)),
        pltpu.SemaphoreType.DMA((_RING,)),
    ],
)
def _agg16(h_hbm, eidx_hbm, out_hbm,
           src_v, dst_v, rows_v, acc, gsem, ssem):
    c = lax.axis_index("c")
    s = lax.axis_index("s")
    w = c * NS + s
    _fill_vmem(rows_v.at[0], CHUNK, 16, 0.0)
    _zero_acc_slice(rows_v.at[0], acc, s)
    pltpu.sync_copy(eidx_hbm.at[0, w], src_v)
    pltpu.sync_copy(eidx_hbm.at[1, w], dst_v)
    plsc.subcore_barrier()
    _agg_body(h_hbm, src_v, dst_v, rows_v, acc, gsem, ssem, NCH_ES,
              _RING, 4)
    plsc.subcore_barrier()
    pltpu.sync_copy(acc.at[pl.ds(s * ROWS_PT, ROWS_PT)],
                    out_hbm.at[c, pl.ds(s * ROWS_PT, ROWS_PT)])


_RB = 1000


def _dis_from_deg(deg_ref):
    deg = jnp.sum(deg_ref[0], axis=1) + jnp.sum(deg_ref[1], axis=1)
    return lax.rsqrt(deg * (1.0 / 16.0) + 1.0)


def _mm1_body(x_ref, w1_ref, deg_ref, h_ref):
    dis = _dis_from_deg(deg_ref)
    h = jnp.dot(x_ref[...], w1_ref[...], preferred_element_type=jnp.float32)
    h = h * dis[:, None]
    h_ref[0] = h[:, :64]
    h_ref[1] = h[:, 64:]


def _mid_body(a_ref, hp_ref, deg_ref, b1_ref, w2_ref, o_ref):
    dis = _dis_from_deg(deg_ref)
    t1a = jnp.tanh(dis[:, None] * (a_ref[0] + hp_ref[0]) + b1_ref[0, :, :])
    t1b = jnp.tanh(dis[:, None] * (a_ref[1] + hp_ref[1]) + b1_ref[1, :, :])
    acc = (jnp.dot(t1a, w2_ref[0], preferred_element_type=jnp.float32)
           + jnp.dot(t1b, w2_ref[1], preferred_element_type=jnp.float32))
    o_ref[...] = dis[:, None] * acc


def _final_body(a_ref, hp_ref, deg_ref, b2_ref, wc_ref, bc_ref, o_ref):
    dis = _dis_from_deg(deg_ref)
    emb = jnp.tanh(dis[:, None] * (a_ref[0] + a_ref[1] + hp_ref[...])
                   + b2_ref[...])
    o_ref[...] = jax.nn.sigmoid(
        jnp.dot(emb, wc_ref[...], preferred_element_type=jnp.float32)
        + bc_ref[...])


def _row_spec(width):
    return pl.BlockSpec((_RB, width), lambda i: (i, 0))


def _pair_spec(width):
    return pl.BlockSpec((NC, _RB, width), lambda i: (0, i, 0))


def _deg_spec(rb):
    return pl.BlockSpec((NC, rb, 16), lambda i: (0, i, 0))


def _full_spec(shape):
    return pl.BlockSpec(shape, lambda i: tuple(0 for _ in shape))


def kernel(x, edge_index, W1, b1, W2, b2, Wc, bc):
    eidx = edge_index.astype(jnp.int32).reshape(2, NC * NS, NCH_ES, CHUNK)

    w2p = jnp.pad(W2, ((0, 0), (0, 16 - W2.shape[1])))
    w2s = w2p.reshape(2, 64, 16)
    wcp = jnp.pad(Wc, ((0, 16 - Wc.shape[0]), (0, 0)))
    b1s = b1.reshape(2, 1, 64)
    b2p = jnp.pad(b2, (0, 16 - b2.shape[0])).reshape(1, 16)
    bcr = bc.reshape(1, 1)

    deg = _deg_kernel(eidx)

    grid = (N // _RB,)
    h1p = pl.pallas_call(
        _mm1_body,
        grid=grid,
        in_specs=[_row_spec(165), _full_spec((165, 128)), _deg_spec(_RB)],
        out_specs=_pair_spec(64),
        out_shape=jax.ShapeDtypeStruct((NC, N, 64), jnp.float32),
    )(x, W1, deg)

    a1 = _agg128(h1p, eidx)

    h2p = pl.pallas_call(
        _mid_body,
        grid=grid,
        in_specs=[_pair_spec(64), _pair_spec(64), _deg_spec(_RB),
                  _full_spec((2, 1, 64)), _full_spec((2, 64, 16))],
        out_specs=_row_spec(16),
        out_shape=jax.ShapeDtypeStruct((N, 16), jnp.float32),
    )(a1, h1p, deg, b1s, w2s)

    a2 = _agg16(h2p, eidx)

    rbf = 2000
    out = pl.pallas_call(
        _final_body,
        grid=(N // rbf,),
        in_specs=[pl.BlockSpec((NC, rbf, 16), lambda i: (0, i, 0)),
                  pl.BlockSpec((rbf, 16), lambda i: (i, 0)),
                  _deg_spec(rbf),
                  _full_spec((1, 16)), _full_spec((16, 1)),
                  _full_spec((1, 1))],
        out_specs=pl.BlockSpec((rbf, 1), lambda i: (i, 0)),
        out_shape=jax.ShapeDtypeStruct((N, 1), jnp.float32),
    )(a2, h2p, deg, b2p, wcp, bcr)

    return out


# speedup vs baseline: 1.1735x; 1.1735x over previous
import functools

import jax
import jax.numpy as jnp
from jax import lax
from jax.experimental import pallas as pl
from jax.experimental.pallas import tpu as pltpu
from jax.experimental.pallas import tpu_sc as plsc

N = 10000
E = 320000
NC = 2
NS = 16
CHUNK = 80
NCH_ES = 125
NCH_FS = 250
NPAD = 10240
ROWS_PT = NPAD // NS

_mesh = plsc.VectorSubcoreMesh(core_axis_name="c", subcore_axis_name="s")
_sc_params = pltpu.CompilerParams(use_tc_tiling_on_sc=False)

_RING = 8


def _fill_vmem(buf, rows, width, value):
    vec = jnp.full((16,), value, jnp.float32)
    for cb in range(width // 16):
        def wr(i, carry):
            buf[i, pl.ds(cb * 16, 16)] = vec
            return carry
        lax.fori_loop(0, rows, wr, 0)


def _zero_acc_slice(zsrc, acc, s):
    for t in range(ROWS_PT // CHUNK):
        pltpu.sync_copy(zsrc, acc.at[pl.ds(s * ROWS_PT + t * CHUNK, CHUNK)])


@functools.partial(
    pl.kernel, mesh=_mesh, compiler_params=_sc_params,
    out_type=jax.ShapeDtypeStruct((NC, NPAD, 16), jnp.float32),
    scratch_types=[
        pltpu.VMEM((NCH_ES, CHUNK), jnp.int32),
        pltpu.VMEM((CHUNK, 16), jnp.float32),
        pltpu.VMEM_SHARED((NPAD, 16), jnp.float32),
        pltpu.SemaphoreType.DMA,
    ],
)
def _deg_kernel(eidx_hbm, out_hbm, idx_v, ones_v, acc, sem):
    c = lax.axis_index("c")
    s = lax.axis_index("s")
    w = c * NS + s
    _fill_vmem(ones_v, CHUNK, 16, 0.0)
    _zero_acc_slice(ones_v, acc, s)
    _fill_vmem(ones_v, CHUNK, 16, 1.0)
    pltpu.sync_copy(eidx_hbm.at[1, w], idx_v)
    plsc.subcore_barrier()

    def fire(j, carry):
        pltpu.async_copy(ones_v, acc.at[idx_v.at[j]], sem, add=True)
        return carry

    lax.fori_loop(0, NCH_ES, fire, 0)

    def drain(j, carry):
        pltpu.make_async_copy(ones_v, acc.at[idx_v.at[j]], sem).wait()
        return carry

    lax.fori_loop(0, NCH_ES, drain, 0)
    plsc.subcore_barrier()
    pltpu.sync_copy(acc.at[pl.ds(s * ROWS_PT, ROWS_PT)],
                    out_hbm.at[c, pl.ds(s * ROWS_PT, ROWS_PT)])


def _agg_body(h_tbl, src_v, dst_v, rows_v, acc, gsem, ssem, nchunk, ring,
              lag):
    def gather_start(j, b):
        pltpu.async_copy(h_tbl.at[src_v.at[j]], rows_v.at[b], gsem.at[b])

    def gather_wait(j, b):
        pltpu.make_async_copy(h_tbl.at[src_v.at[j]], rows_v.at[b],
                              gsem.at[b]).wait()

    def scat_start(j, b):
        pltpu.async_copy(rows_v.at[b], acc.at[dst_v.at[j]], ssem.at[b],
                         add=True)

    def scat_wait(j, b):
        pltpu.make_async_copy(rows_v.at[b], acc.at[dst_v.at[j]],
                              ssem.at[b]).wait()

    for k in range(ring):
        gather_start(k, k)

    def body(j, carry):
        b = lax.rem(j, ring)
        gather_wait(j, b)

        @pl.when((j >= lag) & (j - lag + ring < nchunk))
        def _():
            jp = j - lag
            bp = lax.rem(jp, ring)
            scat_wait(jp, bp)
            gather_start(jp + ring, bp)

        scat_start(j, b)
        return carry

    lax.fori_loop(0, nchunk, body, 0)
    for k in range(nchunk - ring, nchunk):
        scat_wait(k, k % ring)


@functools.partial(
    pl.kernel, mesh=_mesh, compiler_params=_sc_params,
    out_type=jax.ShapeDtypeStruct((NC, NPAD, 64), jnp.float32),
    scratch_types=[
        pltpu.VMEM((NCH_FS, CHUNK), jnp.int32),
        pltpu.VMEM((NCH_FS, CHUNK), jnp.int32),
        pltpu.VMEM((_RING, CHUNK, 64), jnp.float32),
        pltpu.VMEM_SHARED((NPAD, 64), jnp.float32),
        pltpu.SemaphoreType.DMA((_RING,)),
        pltpu.SemaphoreType.DMA((_RING,)),
    ],
)
def _agg128(h_hbm, eidx_hbm, out_hbm,
            src_v, dst_v, rows_v, acc, gsem, ssem):
    c = lax.axis_index("c")
    s = lax.axis_index("s")
    _fill_vmem(rows_v.at[0], CHUNK, 64, 0.0)
    _zero_acc_slice(rows_v.at[0], acc, s)
    pltpu.sync_copy(eidx_hbm.at[0, 2 * s], src_v.at[pl.ds(0, NCH_ES)])
    pltpu.sync_copy(eidx_hbm.at[0, 2 * s + 1], src_v.at[pl.ds(NCH_ES, NCH_ES)])
    pltpu.sync_copy(eidx_hbm.at[1, 2 * s], dst_v.at[pl.ds(0, NCH_ES)])
    pltpu.sync_copy(eidx_hbm.at[1, 2 * s + 1], dst_v.at[pl.ds(NCH_ES, NCH_ES)])
    plsc.subcore_barrier()
    _agg_body(h_hbm.at[c], src_v, dst_v, rows_v, acc, gsem, ssem, NCH_FS,
              _RING, 2)
    plsc.subcore_barrier()
    pltpu.sync_copy(acc.at[pl.ds(s * ROWS_PT, ROWS_PT)],
                    out_hbm.at[c, pl.ds(s * ROWS_PT, ROWS_PT)])


@functools.partial(
    pl.kernel, mesh=_mesh, compiler_params=_sc_params,
    out_type=jax.ShapeDtypeStruct((NC, NPAD, 16), jnp.float32),
    scratch_types=[
        pltpu.VMEM((NCH_ES, CHUNK), jnp.int32),
        pltpu.VMEM((NCH_ES, CHUNK), jnp.int32),
        pltpu.VMEM((_RING, CHUNK, 16), jnp.float32),
        pltpu.VMEM_SHARED((NPAD, 16), jnp.float32),
        pltpu.SemaphoreType.DMA((_RING,)),
        pltpu.SemaphoreType.DMA((_RING,)),
    ],
)
def _agg16(h_hbm, eidx_hbm, out_hbm,
           src_v, dst_v, rows_v, acc, gsem, ssem):
    c = lax.axis_index("c")
    s = lax.axis_index("s")
    w = c * NS + s
    _fill_vmem(rows_v.at[0], CHUNK, 16, 0.0)
    _zero_acc_slice(rows_v.at[0], acc, s)
    pltpu.sync_copy(eidx_hbm.at[0, w], src_v)
    pltpu.sync_copy(eidx_hbm.at[1, w], dst_v)
    plsc.subcore_barrier()
    _agg_body(h_hbm, src_v, dst_v, rows_v, acc, gsem, ssem, NCH_ES,
              _RING, 4)
    plsc.subcore_barrier()
    pltpu.sync_copy(acc.at[pl.ds(s * ROWS_PT, ROWS_PT)],
                    out_hbm.at[c, pl.ds(s * ROWS_PT, ROWS_PT)])


_RB = 1000


def _dis_from_deg(deg_ref):
    deg = jnp.sum(deg_ref[0], axis=1) + jnp.sum(deg_ref[1], axis=1)
    return lax.rsqrt(deg * (1.0 / 16.0) + 1.0)


def _mm1_body(x_ref, w1_ref, deg_ref, h_ref):
    dis = _dis_from_deg(deg_ref)
    h = jnp.dot(x_ref[...], w1_ref[...], preferred_element_type=jnp.float32)
    h = h * dis[:, None]
    h_ref[0] = h[:, :64]
    h_ref[1] = h[:, 64:]


def _mid_body(a_ref, hp_ref, deg_ref, b1_ref, w2_ref, o_ref):
    dis = _dis_from_deg(deg_ref)
    t1a = jnp.tanh(dis[:, None] * (a_ref[0] + hp_ref[0]) + b1_ref[0, :, :])
    t1b = jnp.tanh(dis[:, None] * (a_ref[1] + hp_ref[1]) + b1_ref[1, :, :])
    acc = (jnp.dot(t1a, w2_ref[0], preferred_element_type=jnp.float32)
           + jnp.dot(t1b, w2_ref[1], preferred_element_type=jnp.float32))
    o_ref[...] = dis[:, None] * acc


def _final_body(a_ref, hp_ref, deg_ref, b2_ref, wc_ref, bc_ref, o_ref):
    dis = _dis_from_deg(deg_ref)
    emb = jnp.tanh(dis[:, None] * (a_ref[0] + a_ref[1] + hp_ref[...])
                   + b2_ref[...])
    o_ref[...] = jax.nn.sigmoid(
        jnp.dot(emb, wc_ref[...], preferred_element_type=jnp.float32)
        + bc_ref[...])


def _row_spec(width):
    return pl.BlockSpec((_RB, width), lambda i: (i, 0))


def _pair_spec(width):
    return pl.BlockSpec((NC, _RB, width), lambda i: (0, i, 0))


def _deg_spec(rb):
    return pl.BlockSpec((NC, rb, 16), lambda i: (0, i, 0))


def _full_spec(shape):
    return pl.BlockSpec(shape, lambda i: tuple(0 for _ in shape))


def kernel(x, edge_index, W1, b1, W2, b2, Wc, bc):
    eidx = edge_index.astype(jnp.int32).reshape(2, NC * NS, NCH_ES, CHUNK)

    w2p = jnp.pad(W2, ((0, 0), (0, 16 - W2.shape[1])))
    w2s = w2p.reshape(2, 64, 16)
    wcp = jnp.pad(Wc, ((0, 16 - Wc.shape[0]), (0, 0)))
    b1s = b1.reshape(2, 1, 64)
    b2p = jnp.pad(b2, (0, 16 - b2.shape[0])).reshape(1, 16)
    bcr = bc.reshape(1, 1)

    deg = _deg_kernel(eidx)

    grid = (N // _RB,)
    h1p = pl.pallas_call(
        _mm1_body,
        grid=grid,
        in_specs=[_row_spec(165), _full_spec((165, 128)), _deg_spec(_RB)],
        out_specs=_pair_spec(64),
        out_shape=jax.ShapeDtypeStruct((NC, N, 64), jnp.float32),
    )(x, W1, deg)

    a1 = _agg128(h1p, eidx)

    h2p = pl.pallas_call(
        _mid_body,
        grid=grid,
        in_specs=[_pair_spec(64), _pair_spec(64), _deg_spec(_RB),
                  _full_spec((2, 1, 64)), _full_spec((2, 64, 16))],
        out_specs=_row_spec(16),
        out_shape=jax.ShapeDtypeStruct((N, 16), jnp.float32),
    )(a1, h1p, deg, b1s, w2s)

    a2 = _agg16(h2p, eidx)

    rbf = 2000
    out = pl.pallas_call(
        _final_body,
        grid=(N // rbf,),
        in_specs=[pl.BlockSpec((NC, rbf, 16), lambda i: (0, i, 0)),
                  pl.BlockSpec((rbf, 16), lambda i: (i, 0)),
                  _deg_spec(rbf),
                  _full_spec((1, 16)), _full_spec((16, 1)),
                  _full_spec((1, 1))],
        out_specs=pl.BlockSpec((rbf, 1), lambda i: (i, 0)),
        out_shape=jax.ShapeDtypeStruct((N, 1), jnp.float32),
    )(a2, h2p, deg, b2p, wcp, bcr)

    return out


# speedup vs baseline: 1.2046x; 1.0265x over previous
import functools

import jax
import jax.numpy as jnp
from jax import lax
from jax.experimental import pallas as pl
from jax.experimental.pallas import tpu as pltpu
from jax.experimental.pallas import tpu_sc as plsc

N = 10000
E = 320000
NC = 2
NS = 16
CHUNK = 80
NCH_ES = 125
NCH_FS = 250
NPAD = 10240
ROWS_PT = NPAD // NS

_mesh = plsc.VectorSubcoreMesh(core_axis_name="c", subcore_axis_name="s")
_sc_params = pltpu.CompilerParams(use_tc_tiling_on_sc=False)

_RING = 8


def _fill_vmem(buf, rows, width, value):
    vec = jnp.full((16,), value, jnp.float32)
    for cb in range(width // 16):
        def wr(i, carry):
            buf[i, pl.ds(cb * 16, 16)] = vec
            return carry
        lax.fori_loop(0, rows, wr, 0)


def _zero_acc_slice(zsrc, acc, s):
    for t in range(ROWS_PT // CHUNK):
        pltpu.sync_copy(zsrc, acc.at[pl.ds(s * ROWS_PT + t * CHUNK, CHUNK)])


@functools.partial(
    pl.kernel, mesh=_mesh, compiler_params=_sc_params,
    out_type=jax.ShapeDtypeStruct((NC, NPAD, 16), jnp.float32),
    scratch_types=[
        pltpu.VMEM((NCH_ES, CHUNK), jnp.int32),
        pltpu.VMEM((CHUNK, 16), jnp.float32),
        pltpu.VMEM_SHARED((NPAD, 16), jnp.float32),
        pltpu.SemaphoreType.DMA,
    ],
)
def _deg_kernel(eidx_hbm, out_hbm, idx_v, ones_v, acc, sem):
    c = lax.axis_index("c")
    s = lax.axis_index("s")
    w = c * NS + s
    _fill_vmem(ones_v, CHUNK, 16, 0.0)
    _zero_acc_slice(ones_v, acc, s)
    _fill_vmem(ones_v, CHUNK, 16, 1.0)
    pltpu.sync_copy(eidx_hbm.at[1, w], idx_v)
    plsc.subcore_barrier()

    def fire(j, carry):
        pltpu.async_copy(ones_v, acc.at[idx_v.at[j]], sem, add=True)
        return carry

    lax.fori_loop(0, NCH_ES, fire, 0)

    def drain(j, carry):
        pltpu.make_async_copy(ones_v, acc.at[idx_v.at[j]], sem).wait()
        return carry

    lax.fori_loop(0, NCH_ES, drain, 0)
    plsc.subcore_barrier()
    pltpu.sync_copy(acc.at[pl.ds(s * ROWS_PT, ROWS_PT)],
                    out_hbm.at[c, pl.ds(s * ROWS_PT, ROWS_PT)])


def _agg_body(h_tbl, src_v, dst_v, rows_v, acc, gsem, ssem, nchunk, ring,
              lag):
    def gather_start(j, b):
        pltpu.async_copy(h_tbl.at[src_v.at[j]], rows_v.at[b], gsem.at[b])

    def gather_wait(j, b):
        pltpu.make_async_copy(h_tbl.at[src_v.at[j]], rows_v.at[b],
                              gsem.at[b]).wait()

    def scat_start(j, b):
        pltpu.async_copy(rows_v.at[b], acc.at[dst_v.at[j]], ssem.at[b],
                         add=True)

    def scat_wait(j, b):
        pltpu.make_async_copy(rows_v.at[b], acc.at[dst_v.at[j]],
                              ssem.at[b]).wait()

    for k in range(ring):
        gather_start(k, k)

    def body(j, carry):
        b = lax.rem(j, ring)
        gather_wait(j, b)

        @pl.when((j >= lag) & (j - lag + ring < nchunk))
        def _():
            jp = j - lag
            bp = lax.rem(jp, ring)
            scat_wait(jp, bp)
            gather_start(jp + ring, bp)

        scat_start(j, b)
        return carry

    lax.fori_loop(0, nchunk, body, 0)
    for k in range(nchunk - ring, nchunk):
        scat_wait(k, k % ring)


@functools.partial(
    pl.kernel, mesh=_mesh, compiler_params=_sc_params,
    out_type=jax.ShapeDtypeStruct((NC, NPAD, 64), jnp.float32),
    scratch_types=[
        pltpu.VMEM((NCH_FS, CHUNK), jnp.int32),
        pltpu.VMEM((NCH_FS, CHUNK), jnp.int32),
        pltpu.VMEM((_RING, CHUNK, 64), jnp.float32),
        pltpu.VMEM_SHARED((NPAD, 64), jnp.float32),
        pltpu.SemaphoreType.DMA((_RING,)),
        pltpu.SemaphoreType.DMA((_RING,)),
    ],
)
def _agg128(h_hbm, eidx_hbm, out_hbm,
            src_v, dst_v, rows_v, acc, gsem, ssem):
    c = lax.axis_index("c")
    s = lax.axis_index("s")
    _fill_vmem(rows_v.at[0], CHUNK, 64, 0.0)
    _zero_acc_slice(rows_v.at[0], acc, s)
    pltpu.sync_copy(eidx_hbm.at[0, 2 * s], src_v.at[pl.ds(0, NCH_ES)])
    pltpu.sync_copy(eidx_hbm.at[0, 2 * s + 1], src_v.at[pl.ds(NCH_ES, NCH_ES)])
    pltpu.sync_copy(eidx_hbm.at[1, 2 * s], dst_v.at[pl.ds(0, NCH_ES)])
    pltpu.sync_copy(eidx_hbm.at[1, 2 * s + 1], dst_v.at[pl.ds(NCH_ES, NCH_ES)])
    plsc.subcore_barrier()
    _agg_body(h_hbm.at[c], src_v, dst_v, rows_v, acc, gsem, ssem, NCH_FS,
              _RING, 1)
    plsc.subcore_barrier()
    pltpu.sync_copy(acc.at[pl.ds(s * ROWS_PT, ROWS_PT)],
                    out_hbm.at[c, pl.ds(s * ROWS_PT, ROWS_PT)])


@functools.partial(
    pl.kernel, mesh=_mesh, compiler_params=_sc_params,
    out_type=jax.ShapeDtypeStruct((NC, NPAD, 16), jnp.float32),
    scratch_types=[
        pltpu.VMEM((NCH_ES, CHUNK), jnp.int32),
        pltpu.VMEM((NCH_ES, CHUNK), jnp.int32),
        pltpu.VMEM((_RING, CHUNK, 16), jnp.float32),
        pltpu.VMEM_SHARED((NPAD, 16), jnp.float32),
        pltpu.SemaphoreType.DMA((_RING,)),
        pltpu.SemaphoreType.DMA((_RING,)),
    ],
)
def _agg16(h_hbm, eidx_hbm, out_hbm,
           src_v, dst_v, rows_v, acc, gsem, ssem):
    c = lax.axis_index("c")
    s = lax.axis_index("s")
    w = c * NS + s
    _fill_vmem(rows_v.at[0], CHUNK, 16, 0.0)
    _zero_acc_slice(rows_v.at[0], acc, s)
    pltpu.sync_copy(eidx_hbm.at[0, w], src_v)
    pltpu.sync_copy(eidx_hbm.at[1, w], dst_v)
    plsc.subcore_barrier()
    _agg_body(h_hbm, src_v, dst_v, rows_v, acc, gsem, ssem, NCH_ES,
              _RING, 2)
    plsc.subcore_barrier()
    pltpu.sync_copy(acc.at[pl.ds(s * ROWS_PT, ROWS_PT)],
                    out_hbm.at[c, pl.ds(s * ROWS_PT, ROWS_PT)])


_RB = 1000


def _dis_from_deg(deg_ref):
    deg = jnp.sum(deg_ref[0], axis=1) + jnp.sum(deg_ref[1], axis=1)
    return lax.rsqrt(deg * (1.0 / 16.0) + 1.0)


def _mm1_body(x_ref, w1_ref, deg_ref, h_ref):
    dis = _dis_from_deg(deg_ref)
    h = jnp.dot(x_ref[...], w1_ref[...], preferred_element_type=jnp.float32)
    h = h * dis[:, None]
    h_ref[0] = h[:, :64]
    h_ref[1] = h[:, 64:]


def _mid_body(a_ref, hp_ref, deg_ref, b1_ref, w2_ref, o_ref):
    dis = _dis_from_deg(deg_ref)
    t1a = jnp.tanh(dis[:, None] * (a_ref[0] + hp_ref[0]) + b1_ref[0, :, :])
    t1b = jnp.tanh(dis[:, None] * (a_ref[1] + hp_ref[1]) + b1_ref[1, :, :])
    acc = (jnp.dot(t1a, w2_ref[0], preferred_element_type=jnp.float32)
           + jnp.dot(t1b, w2_ref[1], preferred_element_type=jnp.float32))
    o_ref[...] = dis[:, None] * acc


def _final_body(a_ref, hp_ref, deg_ref, b2_ref, wc_ref, bc_ref, o_ref):
    dis = _dis_from_deg(deg_ref)
    emb = jnp.tanh(dis[:, None] * (a_ref[0] + a_ref[1] + hp_ref[...])
                   + b2_ref[...])
    o_ref[...] = jax.nn.sigmoid(
        jnp.dot(emb, wc_ref[...], preferred_element_type=jnp.float32)
        + bc_ref[...])


def _row_spec(width):
    return pl.BlockSpec((_RB, width), lambda i: (i, 0))


def _pair_spec(width):
    return pl.BlockSpec((NC, _RB, width), lambda i: (0, i, 0))


def _deg_spec(rb):
    return pl.BlockSpec((NC, rb, 16), lambda i: (0, i, 0))


def _full_spec(shape):
    return pl.BlockSpec(shape, lambda i: tuple(0 for _ in shape))


def kernel(x, edge_index, W1, b1, W2, b2, Wc, bc):
    eidx = edge_index.astype(jnp.int32).reshape(2, NC * NS, NCH_ES, CHUNK)

    w2p = jnp.pad(W2, ((0, 0), (0, 16 - W2.shape[1])))
    w2s = w2p.reshape(2, 64, 16)
    wcp = jnp.pad(Wc, ((0, 16 - Wc.shape[0]), (0, 0)))
    b1s = b1.reshape(2, 1, 64)
    b2p = jnp.pad(b2, (0, 16 - b2.shape[0])).reshape(1, 16)
    bcr = bc.reshape(1, 1)

    deg = _deg_kernel(eidx)

    grid = (N // _RB,)
    h1p = pl.pallas_call(
        _mm1_body,
        grid=grid,
        in_specs=[_row_spec(165), _full_spec((165, 128)), _deg_spec(_RB)],
        out_specs=_pair_spec(64),
        out_shape=jax.ShapeDtypeStruct((NC, N, 64), jnp.float32),
    )(x, W1, deg)

    a1 = _agg128(h1p, eidx)

    h2p = pl.pallas_call(
        _mid_body,
        grid=grid,
        in_specs=[_pair_spec(64), _pair_spec(64), _deg_spec(_RB),
                  _full_spec((2, 1, 64)), _full_spec((2, 64, 16))],
        out_specs=_row_spec(16),
        out_shape=jax.ShapeDtypeStruct((N, 16), jnp.float32),
    )(a1, h1p, deg, b1s, w2s)

    a2 = _agg16(h2p, eidx)

    rbf = 2000
    out = pl.pallas_call(
        _final_body,
        grid=(N // rbf,),
        in_specs=[pl.BlockSpec((NC, rbf, 16), lambda i: (0, i, 0)),
                  pl.BlockSpec((rbf, 16), lambda i: (i, 0)),
                  _deg_spec(rbf),
                  _full_spec((1, 16)), _full_spec((16, 1)),
                  _full_spec((1, 1))],
        out_specs=pl.BlockSpec((rbf, 1), lambda i: (i, 0)),
        out_shape=jax.ShapeDtypeStruct((N, 1), jnp.float32),
    )(a2, h2p, deg, b2p, wcp, bcr)

    return out


# speedup vs baseline: 1.2214x; 1.0140x over previous
import functools

import jax
import jax.numpy as jnp
from jax import lax
from jax.experimental import pallas as pl
from jax.experimental.pallas import tpu as pltpu
from jax.experimental.pallas import tpu_sc as plsc

N = 10000
E = 320000
NC = 2
NS = 16
CHUNK = 80
NCH_ES = 125
NCH_FS = 250
NPAD = 10240
ROWS_PT = NPAD // NS

_mesh = plsc.VectorSubcoreMesh(core_axis_name="c", subcore_axis_name="s")
_sc_params = pltpu.CompilerParams(use_tc_tiling_on_sc=False)

_RING = 8
_RING16 = 12


def _fill_vmem(buf, rows, width, value):
    vec = jnp.full((16,), value, jnp.float32)
    for cb in range(width // 16):
        def wr(i, carry):
            buf[i, pl.ds(cb * 16, 16)] = vec
            return carry
        lax.fori_loop(0, rows, wr, 0)


def _zero_acc_slice(zsrc, acc, s):
    for t in range(ROWS_PT // CHUNK):
        pltpu.sync_copy(zsrc, acc.at[pl.ds(s * ROWS_PT + t * CHUNK, CHUNK)])


@functools.partial(
    pl.kernel, mesh=_mesh, compiler_params=_sc_params,
    out_type=jax.ShapeDtypeStruct((NC, NPAD, 16), jnp.float32),
    scratch_types=[
        pltpu.VMEM((NCH_ES, CHUNK), jnp.int32),
        pltpu.VMEM((CHUNK, 16), jnp.float32),
        pltpu.VMEM_SHARED((NPAD, 16), jnp.float32),
        pltpu.SemaphoreType.DMA,
    ],
)
def _deg_kernel(eidx_hbm, out_hbm, idx_v, ones_v, acc, sem):
    c = lax.axis_index("c")
    s = lax.axis_index("s")
    w = c * NS + s
    _fill_vmem(ones_v, CHUNK, 16, 0.0)
    _zero_acc_slice(ones_v, acc, s)
    _fill_vmem(ones_v, CHUNK, 16, 1.0)
    pltpu.sync_copy(eidx_hbm.at[1, w], idx_v)
    plsc.subcore_barrier()

    def fire(j, carry):
        pltpu.async_copy(ones_v, acc.at[idx_v.at[j]], sem, add=True)
        return carry

    lax.fori_loop(0, NCH_ES, fire, 0)

    def drain(j, carry):
        pltpu.make_async_copy(ones_v, acc.at[idx_v.at[j]], sem).wait()
        return carry

    lax.fori_loop(0, NCH_ES, drain, 0)
    plsc.subcore_barrier()
    pltpu.sync_copy(acc.at[pl.ds(s * ROWS_PT, ROWS_PT)],
                    out_hbm.at[c, pl.ds(s * ROWS_PT, ROWS_PT)])


def _agg_body(h_tbl, src_v, dst_v, rows_v, acc, gsem, ssem, nchunk, ring,
              lag):
    def gather_start(j, b):
        pltpu.async_copy(h_tbl.at[src_v.at[j]], rows_v.at[b], gsem.at[b])

    def gather_wait(j, b):
        pltpu.make_async_copy(h_tbl.at[src_v.at[j]], rows_v.at[b],
                              gsem.at[b]).wait()

    def scat_start(j, b):
        pltpu.async_copy(rows_v.at[b], acc.at[dst_v.at[j]], ssem.at[b],
                         add=True)

    def scat_wait(j, b):
        pltpu.make_async_copy(rows_v.at[b], acc.at[dst_v.at[j]],
                              ssem.at[b]).wait()

    for k in range(ring):
        gather_start(k, k)

    def body(j, carry):
        b = lax.rem(j, ring)
        gather_wait(j, b)

        @pl.when((j >= lag) & (j - lag + ring < nchunk))
        def _():
            jp = j - lag
            bp = lax.rem(jp, ring)
            scat_wait(jp, bp)
            gather_start(jp + ring, bp)

        scat_start(j, b)
        return carry

    lax.fori_loop(0, nchunk, body, 0)
    for k in range(nchunk - ring, nchunk):
        scat_wait(k, k % ring)


@functools.partial(
    pl.kernel, mesh=_mesh, compiler_params=_sc_params,
    out_type=jax.ShapeDtypeStruct((NC, NPAD, 64), jnp.float32),
    scratch_types=[
        pltpu.VMEM((NCH_FS, CHUNK), jnp.int32),
        pltpu.VMEM((NCH_FS, CHUNK), jnp.int32),
        pltpu.VMEM((_RING, CHUNK, 64), jnp.float32),
        pltpu.VMEM_SHARED((NPAD, 64), jnp.float32),
        pltpu.SemaphoreType.DMA((_RING,)),
        pltpu.SemaphoreType.DMA((_RING,)),
    ],
)
def _agg128(h_hbm, eidx_hbm, out_hbm,
            src_v, dst_v, rows_v, acc, gsem, ssem):
    c = lax.axis_index("c")
    s = lax.axis_index("s")
    _fill_vmem(rows_v.at[0], CHUNK, 64, 0.0)
    _zero_acc_slice(rows_v.at[0], acc, s)
    pltpu.sync_copy(eidx_hbm.at[0, 2 * s], src_v.at[pl.ds(0, NCH_ES)])
    pltpu.sync_copy(eidx_hbm.at[0, 2 * s + 1], src_v.at[pl.ds(NCH_ES, NCH_ES)])
    pltpu.sync_copy(eidx_hbm.at[1, 2 * s], dst_v.at[pl.ds(0, NCH_ES)])
    pltpu.sync_copy(eidx_hbm.at[1, 2 * s + 1], dst_v.at[pl.ds(NCH_ES, NCH_ES)])
    plsc.subcore_barrier()
    _agg_body(h_hbm.at[c], src_v, dst_v, rows_v, acc, gsem, ssem, NCH_FS,
              _RING, 1)
    plsc.subcore_barrier()
    pltpu.sync_copy(acc.at[pl.ds(s * ROWS_PT, ROWS_PT)],
                    out_hbm.at[c, pl.ds(s * ROWS_PT, ROWS_PT)])


@functools.partial(
    pl.kernel, mesh=_mesh, compiler_params=_sc_params,
    out_type=jax.ShapeDtypeStruct((NC, NPAD, 16), jnp.float32),
    scratch_types=[
        pltpu.VMEM((NCH_ES, CHUNK), jnp.int32),
        pltpu.VMEM((NCH_ES, CHUNK), jnp.int32),
        pltpu.VMEM((_RING16, CHUNK, 16), jnp.float32),
        pltpu.VMEM_SHARED((NPAD, 16), jnp.float32),
        pltpu.SemaphoreType.DMA((_RING16,)),
        pltpu.SemaphoreType.DMA((_RING16,)),
    ],
)
def _agg16(h_hbm, eidx_hbm, out_hbm,
           src_v, dst_v, rows_v, acc, gsem, ssem):
    c = lax.axis_index("c")
    s = lax.axis_index("s")
    w = c * NS + s
    _fill_vmem(rows_v.at[0], CHUNK, 16, 0.0)
    _zero_acc_slice(rows_v.at[0], acc, s)
    pltpu.sync_copy(eidx_hbm.at[0, w], src_v)
    pltpu.sync_copy(eidx_hbm.at[1, w], dst_v)
    plsc.subcore_barrier()
    _agg_body(h_hbm, src_v, dst_v, rows_v, acc, gsem, ssem, NCH_ES,
              _RING16, 1)
    plsc.subcore_barrier()
    pltpu.sync_copy(acc.at[pl.ds(s * ROWS_PT, ROWS_PT)],
                    out_hbm.at[c, pl.ds(s * ROWS_PT, ROWS_PT)])


_RB = 1000


def _dis_from_deg(deg_ref):
    deg = jnp.sum(deg_ref[0], axis=1) + jnp.sum(deg_ref[1], axis=1)
    return lax.rsqrt(deg * (1.0 / 16.0) + 1.0)


def _mm1_body(x_ref, w1_ref, deg_ref, h_ref):
    dis = _dis_from_deg(deg_ref)
    h = jnp.dot(x_ref[...], w1_ref[...], preferred_element_type=jnp.float32)
    h = h * dis[:, None]
    h_ref[0] = h[:, :64]
    h_ref[1] = h[:, 64:]


def _mid_body(a_ref, hp_ref, deg_ref, b1_ref, w2_ref, o_ref):
    dis = _dis_from_deg(deg_ref)
    t1a = jnp.tanh(dis[:, None] * (a_ref[0] + hp_ref[0]) + b1_ref[0, :, :])
    t1b = jnp.tanh(dis[:, None] * (a_ref[1] + hp_ref[1]) + b1_ref[1, :, :])
    acc = (jnp.dot(t1a, w2_ref[0], preferred_element_type=jnp.float32)
           + jnp.dot(t1b, w2_ref[1], preferred_element_type=jnp.float32))
    o_ref[...] = dis[:, None] * acc


def _final_body(a_ref, hp_ref, deg_ref, b2_ref, wc_ref, bc_ref, o_ref):
    dis = _dis_from_deg(deg_ref)
    emb = jnp.tanh(dis[:, None] * (a_ref[0] + a_ref[1] + hp_ref[...])
                   + b2_ref[...])
    o_ref[...] = jax.nn.sigmoid(
        jnp.dot(emb, wc_ref[...], preferred_element_type=jnp.float32)
        + bc_ref[...])


def _row_spec(width):
    return pl.BlockSpec((_RB, width), lambda i: (i, 0))


def _pair_spec(width):
    return pl.BlockSpec((NC, _RB, width), lambda i: (0, i, 0))


def _deg_spec(rb):
    return pl.BlockSpec((NC, rb, 16), lambda i: (0, i, 0))


def _full_spec(shape):
    return pl.BlockSpec(shape, lambda i: tuple(0 for _ in shape))


def kernel(x, edge_index, W1, b1, W2, b2, Wc, bc):
    eidx = edge_index.astype(jnp.int32).reshape(2, NC * NS, NCH_ES, CHUNK)

    w2p = jnp.pad(W2, ((0, 0), (0, 16 - W2.shape[1])))
    w2s = w2p.reshape(2, 64, 16)
    wcp = jnp.pad(Wc, ((0, 16 - Wc.shape[0]), (0, 0)))
    b1s = b1.reshape(2, 1, 64)
    b2p = jnp.pad(b2, (0, 16 - b2.shape[0])).reshape(1, 16)
    bcr = bc.reshape(1, 1)

    deg = _deg_kernel(eidx)

    grid = (N // _RB,)
    h1p = pl.pallas_call(
        _mm1_body,
        grid=grid,
        in_specs=[_row_spec(165), _full_spec((165, 128)), _deg_spec(_RB)],
        out_specs=_pair_spec(64),
        out_shape=jax.ShapeDtypeStruct((NC, N, 64), jnp.float32),
    )(x, W1, deg)

    a1 = _agg128(h1p, eidx)

    h2p = pl.pallas_call(
        _mid_body,
        grid=grid,
        in_specs=[_pair_spec(64), _pair_spec(64), _deg_spec(_RB),
                  _full_spec((2, 1, 64)), _full_spec((2, 64, 16))],
        out_specs=_row_spec(16),
        out_shape=jax.ShapeDtypeStruct((N, 16), jnp.float32),
    )(a1, h1p, deg, b1s, w2s)

    a2 = _agg16(h2p, eidx)

    rbf = 2000
    out = pl.pallas_call(
        _final_body,
        grid=(N // rbf,),
        in_specs=[pl.BlockSpec((NC, rbf, 16), lambda i: (0, i, 0)),
                  pl.BlockSpec((rbf, 16), lambda i: (i, 0)),
                  _deg_spec(rbf),
                  _full_spec((1, 16)), _full_spec((16, 1)),
                  _full_spec((1, 1))],
        out_specs=pl.BlockSpec((rbf, 1), lambda i: (i, 0)),
        out_shape=jax.ShapeDtypeStruct((N, 1), jnp.float32),
    )(a2, h2p, deg, b2p, wcp, bcr)

    return out
